# Initial kernel scaffold; baseline (speedup 1.0000x reference)
import jax
import jax.numpy as jnp
from jax.experimental import pallas as pl


def kernel(x, Wg1, bg1, Wg2, bg2, We1, be1, We2, be2):
    raise NotImplementedError("write your pallas kernel here")


# speedup vs baseline: 1.0170x; 1.0170x over previous
import functools

import jax
import jax.numpy as jnp
from jax.experimental import pallas as pl
from jax.experimental.pallas import tpu as pltpu
from jax.experimental.pallas import tpu_sc as plsc


_N_SPLIT = 8


def _gating_body(x_ref, wg1_ref, bg1_ref, wg2_ref, bg2_ref,
                 gate_out_ref, idx_ref, rows_ref):
    x = x_ref[...]
    h = jax.lax.dot(x.astype(jnp.bfloat16), wg1_ref[...].astype(jnp.bfloat16),
                    preferred_element_type=jnp.float32) + bg1_ref[...]
    h = jnp.maximum(h, 0.0)
    g = jax.lax.dot(h.astype(jnp.bfloat16), wg2_ref[...].astype(jnp.bfloat16),
                    preferred_element_type=jnp.float32) + bg2_ref[...]
    idx = jnp.argmax(g, axis=1).astype(jnp.int32)
    e_iota = jax.lax.broadcasted_iota(jnp.int32, g.shape, 1)
    gate_out_ref[...] = (e_iota == idx[:, None]).astype(jnp.float32)
    idx_ref[...] = idx[:, None]
    n_split = rows_ref.shape[1]
    b_iota = jax.lax.broadcasted_iota(jnp.int32, rows_ref.shape, 0)
    j_iota = jax.lax.broadcasted_iota(jnp.int32, rows_ref.shape, 1)
    rows_ref[...] = (idx[:, None] * g.shape[0] + b_iota) * n_split + j_iota


def _gating(x, Wg1, bg1, Wg2, bg2):
    B = x.shape[0]
    E = Wg2.shape[1]
    gate_out, idx, rows = pl.pallas_call(
        _gating_body,
        out_shape=[
            jax.ShapeDtypeStruct((B, E), jnp.float32),
            jax.ShapeDtypeStruct((B, 1), jnp.int32),
            jax.ShapeDtypeStruct((B, _N_SPLIT), jnp.int32),
        ],
    )(x, Wg1, bg1.reshape(1, -1), Wg2, bg2.reshape(1, -1))
    return gate_out, idx, rows


def _expert_body(x_ref, we1_ref, be1_ref, we2_ref, be2_ref, out_ref):
    x = x_ref[...].astype(jnp.bfloat16)
    w1 = we1_ref[0].astype(jnp.bfloat16)
    h = jax.lax.dot(x, w1, preferred_element_type=jnp.float32)
    h = jnp.maximum(h + be1_ref[0], 0.0)
    o = jax.lax.dot(h.astype(jnp.bfloat16), we2_ref[0].astype(jnp.bfloat16),
                    preferred_element_type=jnp.float32)
    o = o + be2_ref[0]
    m = jnp.max(o, axis=1, keepdims=True)
    eo = jnp.exp(o - m)
    p = eo / jnp.sum(eo, axis=1, keepdims=True)
    out_ref[...] = p.reshape(out_ref.shape)


def _experts(x, We1, be1, We2, be2, bm):
    B, D = x.shape
    E, _, H = We1.shape
    C = We2.shape[2]
    nb = B // bm
    ns = _N_SPLIT
    out = pl.pallas_call(
        _expert_body,
        grid=(E, nb),
        in_specs=[
            pl.BlockSpec((bm, D), lambda e, i: (i, 0)),
            pl.BlockSpec((1, D, H), lambda e, i: (e, 0, 0)),
            pl.BlockSpec((1, 1, H), lambda e, i: (e, 0, 0)),
            pl.BlockSpec((1, H, C), lambda e, i: (e, 0, 0)),
            pl.BlockSpec((1, 1, C), lambda e, i: (e, 0, 0)),
        ],
        out_specs=pl.BlockSpec((bm * ns, C // ns),
                               lambda e, i: (e * (B // bm) + i, 0)),
        out_shape=jax.ShapeDtypeStruct((E * B * ns, C // ns), jnp.float32),
        compiler_params=pltpu.CompilerParams(
            dimension_semantics=("arbitrary", "arbitrary"),
        ),
    )(x, We1, be1.reshape(E, 1, H), We2, be2.reshape(E, 1, C))
    return out


def _sc_gather(eo_rows, rows, window):
    B = rows.shape[1]
    C = eo_rows.shape[1]
    mesh = plsc.VectorSubcoreMesh(core_axis_name="core",
                                  subcore_axis_name="subcore")

    @pl.kernel(out_type=jax.ShapeDtypeStruct((B, C), eo_rows.dtype),
               mesh=mesh)
    def kern(eo_hbm, rows_hbm, o_hbm):
        def body(i_vmem, o_vmem):
            pltpu.sync_copy(eo_hbm.at[i_vmem.at[0]], o_vmem)

        pltpu.emit_pipeline(
            body,
            grid=(B // window,),
            in_specs=[pl.BlockSpec((1, window), index_map=lambda i: (0, i))],
            out_specs=[pl.BlockSpec((window, C), index_map=lambda i: (i, 0))],
            core_axis_name=("core", "subcore"),
            dimension_semantics=(pltpu.PARALLEL,),
        )(rows_hbm, o_hbm)

    return kern(eo_rows, rows)


@jax.jit
def kernel(x, Wg1, bg1, Wg2, bg2, We1, be1, We2, be2):
    B, D = x.shape
    E, _, H = We1.shape
    C = We2.shape[2]

    gate_outputs, idx2d, rows2d = _gating(x, Wg1, bg1, Wg2, bg2)
    eo_lin = _experts(x, We1, be1, We2, be2, bm=512)
    expert_outputs = eo_lin.reshape(E, B, C).transpose(1, 0, 2)
    final_output = _sc_gather(
        eo_lin, rows2d.reshape(1, B * _N_SPLIT), window=128,
    ).reshape(B, C)
    expert_indices = idx2d.reshape(B)
    return final_output, expert_outputs, gate_outputs, expert_indices


# speedup vs baseline: 1.2477x; 1.2269x over previous
import functools

import jax
import jax.numpy as jnp
from jax.experimental import pallas as pl
from jax.experimental.pallas import tpu as pltpu
from jax.experimental.pallas import tpu_sc as plsc


_N_SPLIT = 8


def _gating_body(x_ref, wg1_ref, bg1_ref, wg2_ref, bg2_ref,
                 gate_out_ref, idx_ref, rows_ref):
    x = x_ref[...]
    h = jax.lax.dot(x.astype(jnp.bfloat16), wg1_ref[...].astype(jnp.bfloat16),
                    preferred_element_type=jnp.float32) + bg1_ref[...]
    h = jnp.maximum(h, 0.0)
    g = jax.lax.dot(h.astype(jnp.bfloat16), wg2_ref[...].astype(jnp.bfloat16),
                    preferred_element_type=jnp.float32) + bg2_ref[...]
    idx = jnp.argmax(g, axis=1).astype(jnp.int32)
    e_iota = jax.lax.broadcasted_iota(jnp.int32, g.shape, 1)
    gate_out_ref[...] = (e_iota == idx[:, None]).astype(jnp.float32)
    idx_ref[...] = idx[:, None]
    n_split = rows_ref.shape[1]
    b_iota = jax.lax.broadcasted_iota(jnp.int32, rows_ref.shape, 0)
    j_iota = jax.lax.broadcasted_iota(jnp.int32, rows_ref.shape, 1)
    rows_ref[...] = (idx[:, None] * g.shape[0] + b_iota) * n_split + j_iota


def _gating(x, Wg1, bg1, Wg2, bg2):
    B = x.shape[0]
    E = Wg2.shape[1]
    gate_out, idx, rows = pl.pallas_call(
        _gating_body,
        out_shape=[
            jax.ShapeDtypeStruct((B, E), jnp.float32),
            jax.ShapeDtypeStruct((B, 1), jnp.int32),
            jax.ShapeDtypeStruct((B, _N_SPLIT), jnp.int32),
        ],
    )(x, Wg1, bg1.reshape(1, -1), Wg2, bg2.reshape(1, -1))
    return gate_out, idx, rows


def _expert_body(x_ref, we1_ref, be1_ref, we2_ref, be2_ref, out_ref):
    x = x_ref[...].astype(jnp.bfloat16)
    w1 = we1_ref[0].astype(jnp.bfloat16)
    h = jax.lax.dot(x, w1, preferred_element_type=jnp.float32)
    h = jnp.maximum(h + be1_ref[0], 0.0)
    o = jax.lax.dot(h.astype(jnp.bfloat16), we2_ref[0].astype(jnp.bfloat16),
                    preferred_element_type=jnp.float32)
    o = o + be2_ref[0]
    m = jnp.max(o, axis=1, keepdims=True)
    eo = jnp.exp(o - m)
    p = eo / jnp.sum(eo, axis=1, keepdims=True)
    out_ref[...] = p.reshape(out_ref.shape)


def _experts(x, We1, be1, We2, be2, bm):
    B, D = x.shape
    E, _, H = We1.shape
    C = We2.shape[2]
    nb = B // bm
    ns = _N_SPLIT
    out = pl.pallas_call(
        _expert_body,
        grid=(E, nb),
        in_specs=[
            pl.BlockSpec((bm, D), lambda e, i: (i, 0)),
            pl.BlockSpec((1, D, H), lambda e, i: (e, 0, 0)),
            pl.BlockSpec((1, 1, H), lambda e, i: (e, 0, 0)),
            pl.BlockSpec((1, H, C), lambda e, i: (e, 0, 0)),
            pl.BlockSpec((1, 1, C), lambda e, i: (e, 0, 0)),
        ],
        out_specs=pl.BlockSpec((bm * ns, C // ns),
                               lambda e, i: (e * (B // bm) + i, 0)),
        out_shape=jax.ShapeDtypeStruct((E * B * ns, C // ns), jnp.float32),
        compiler_params=pltpu.CompilerParams(
            dimension_semantics=("arbitrary", "arbitrary"),
        ),
    )(x, We1, be1.reshape(E, 1, H), We2, be2.reshape(E, 1, C))
    return out


def _interleave_body(*refs):
    in_refs, out_ref = refs[:-1], refs[-1]
    bm = out_ref.shape[0]
    c = out_ref.shape[2]
    for j, r in enumerate(in_refs):
        out_ref[:, j, :] = r[...].reshape(bm, c)


def _interleave(eo_lin, B, E, C, bm):
    ns = _N_SPLIT
    nb = B // bm

    def mk_map(j):
        return lambda i: (j * nb + i, 0)

    return pl.pallas_call(
        _interleave_body,
        grid=(nb,),
        in_specs=[pl.BlockSpec((bm * ns, C // ns), mk_map(j))
                  for j in range(E)],
        out_specs=pl.BlockSpec((bm, E, C), lambda i: (i, 0, 0)),
        out_shape=jax.ShapeDtypeStruct((B, E, C), jnp.float32),
        compiler_params=pltpu.CompilerParams(
            dimension_semantics=("arbitrary",),
        ),
    )(*([eo_lin] * E))


def _sc_gather(eo_rows, rows, window):
    B = rows.shape[1]
    C = eo_rows.shape[1]
    mesh = plsc.VectorSubcoreMesh(core_axis_name="core",
                                  subcore_axis_name="subcore")

    @pl.kernel(out_type=jax.ShapeDtypeStruct((B, C), eo_rows.dtype),
               mesh=mesh)
    def kern(eo_hbm, rows_hbm, o_hbm):
        def body(i_vmem, o_vmem):
            pltpu.sync_copy(eo_hbm.at[i_vmem.at[0]], o_vmem)

        pltpu.emit_pipeline(
            body,
            grid=(B // window,),
            in_specs=[pl.BlockSpec((1, window), index_map=lambda i: (0, i))],
            out_specs=[pl.BlockSpec((window, C), index_map=lambda i: (i, 0))],
            core_axis_name=("core", "subcore"),
            dimension_semantics=(pltpu.PARALLEL,),
        )(rows_hbm, o_hbm)

    return kern(eo_rows, rows)


@jax.jit
def kernel(x, Wg1, bg1, Wg2, bg2, We1, be1, We2, be2):
    B, D = x.shape
    E, _, H = We1.shape
    C = We2.shape[2]

    gate_outputs, idx2d, rows2d = _gating(x, Wg1, bg1, Wg2, bg2)
    eo_lin = _experts(x, We1, be1, We2, be2, bm=512)
    expert_outputs = _interleave(eo_lin, B, E, C, bm=256)
    final_output = _sc_gather(
        eo_lin, rows2d.reshape(1, B * _N_SPLIT), window=128,
    ).reshape(B, C)
    expert_indices = idx2d.reshape(B)
    return final_output, expert_outputs, gate_outputs, expert_indices
